# Initial kernel scaffold; baseline (speedup 1.0000x reference)
import jax
import jax.numpy as jnp
from jax.experimental import pallas as pl


def kernel(inputs, table, W_out, b_out):
    raise NotImplementedError("write your pallas kernel here")


# speedup vs baseline: 9.9789x; 9.9789x over previous
import functools

import jax
import jax.numpy as jnp
from jax import lax
from jax.experimental import pallas as pl
from jax.experimental.pallas import tpu as pltpu
from jax.experimental.pallas import tpu_sc as plsc

_NC = 2
_NS = 16
_NW = _NC * _NS
_LANES = 16


def _make_pool_kernel(B, SP, D):
    BPW = B // _NW
    HC = SP // 2
    ND = D // _LANES

    mesh = plsc.VectorSubcoreMesh(core_axis_name="c", subcore_axis_name="s")

    @functools.partial(
        pl.kernel,
        mesh=mesh,
        out_type=jax.ShapeDtypeStruct((B, D), jnp.float32),
        compiler_params=pltpu.CompilerParams(use_tc_tiling_on_sc=False),
        scratch_types=[
            pltpu.VMEM((BPW, 2, HC), jnp.int32),
            pltpu.VMEM((2, HC, D), jnp.float32),
            pltpu.VMEM((BPW, D), jnp.float32),
            pltpu.SemaphoreType.DMA,
            pltpu.SemaphoreType.DMA,
        ],
    )
    def pool(idx_hbm, table_hbm, out_hbm, idx_v, rows_v, pool_v, sem0, sem1):
        cid = lax.axis_index("c")
        sid = lax.axis_index("s")
        wid = sid * _NC + cid
        base = wid * BPW

        pltpu.sync_copy(idx_hbm.at[pl.ds(base, BPW)], idx_v)

        def reduce_chunk(c, accs):
            def rbody(r, accs):
                return tuple(
                    jnp.maximum(accs[k], rows_v[c, r, pl.ds(k * _LANES, _LANES)])
                    for k in range(ND)
                )
            return lax.fori_loop(0, HC, rbody, accs, unroll=4)

        def body(b, _):
            cp0 = pltpu.make_async_copy(
                table_hbm.at[idx_v.at[b, 0]], rows_v.at[0], sem0)
            cp1 = pltpu.make_async_copy(
                table_hbm.at[idx_v.at[b, 1]], rows_v.at[1], sem1)
            cp0.start()
            cp1.start()
            cp0.wait()
            accs = tuple(
                jnp.full((_LANES,), -jnp.inf, jnp.float32) for _ in range(ND))
            accs = reduce_chunk(0, accs)
            cp1.wait()
            accs = reduce_chunk(1, accs)
            for k in range(ND):
                pool_v[b, pl.ds(k * _LANES, _LANES)] = accs[k]
            return 0

        lax.fori_loop(0, BPW, body, 0)

        pltpu.sync_copy(pool_v, out_hbm.at[pl.ds(base, BPW)])

    return pool


def _dense_softmax(pooled, w_pad, b_pad, n_out):
    B, D = pooled.shape
    BLK = 512

    def body(x_ref, w_ref, b_ref, o_ref):
        x = x_ref[...]
        logits = jnp.dot(x, w_ref[...], preferred_element_type=jnp.float32)
        logits = logits + b_ref[...]
        mask = lax.broadcasted_iota(jnp.int32, logits.shape, 1) < n_out
        masked = jnp.where(mask, logits, -jnp.inf)
        m = jnp.max(masked, axis=1, keepdims=True)
        e = jnp.where(mask, jnp.exp(masked - m), 0.0)
        o_ref[...] = e / jnp.sum(e, axis=1, keepdims=True)

    out = pl.pallas_call(
        body,
        grid=(B // BLK,),
        in_specs=[
            pl.BlockSpec((BLK, D), lambda i: (i, 0)),
            pl.BlockSpec((D, 128), lambda i: (0, 0)),
            pl.BlockSpec((1, 128), lambda i: (0, 0)),
        ],
        out_specs=pl.BlockSpec((BLK, 128), lambda i: (i, 0)),
        out_shape=jax.ShapeDtypeStruct((B, 128), jnp.float32),
    )(pooled, w_pad, b_pad)
    return out[:, :n_out]


def kernel(inputs, table, W_out, b_out):
    B, S = inputs.shape
    V, D = table.shape
    n_out = W_out.shape[1]

    SP = ((S + 15) // 16) * 16
    idx = inputs.astype(jnp.int32)
    if SP != S:
        idx = jnp.concatenate([idx, idx[:, : SP - S]], axis=1)
    idx = idx.reshape(B, 2, SP // 2)

    pooled = _make_pool_kernel(B, SP, D)(idx, table)

    w_pad = jnp.zeros((D, 128), jnp.float32).at[:, :n_out].set(W_out)
    b_pad = jnp.zeros((1, 128), jnp.float32).at[0, :n_out].set(b_out)
    return _dense_softmax(pooled, w_pad, b_pad, n_out)


# speedup vs baseline: 14.0408x; 1.4070x over previous
import functools

import jax
import jax.numpy as jnp
from jax import lax
from jax.experimental import pallas as pl
from jax.experimental.pallas import tpu as pltpu
from jax.experimental.pallas import tpu_sc as plsc

_NC = 2
_NS = 16
_NW = _NC * _NS
_LANES = 16


def _make_pool_kernel(B, S, D):
    BPW = B // _NW
    C0 = 104
    C1 = S - C0
    ND = D // _LANES

    mesh = plsc.VectorSubcoreMesh(core_axis_name="c", subcore_axis_name="s")

    @functools.partial(
        pl.kernel,
        mesh=mesh,
        out_type=jax.ShapeDtypeStruct((B, D), jnp.float32),
        compiler_params=pltpu.CompilerParams(use_tc_tiling_on_sc=False),
        scratch_types=[
            pltpu.VMEM((BPW, C0), jnp.int32),
            pltpu.VMEM((BPW, C1), jnp.int32),
            pltpu.VMEM((2, C0, D), jnp.float32),
            pltpu.VMEM((2, C1, D), jnp.float32),
            pltpu.VMEM((BPW, D), jnp.float32),
            pltpu.SemaphoreType.DMA,
            pltpu.SemaphoreType.DMA,
        ],
    )
    def pool(idx_hbm, table_hbm, out_hbm, idx0_v, idx1_v, rows0_v, rows1_v,
             pool_v, sem0, sem1):
        cid = lax.axis_index("c")
        sid = lax.axis_index("s")
        wid = sid * _NC + cid
        base = wid * BPW

        pltpu.sync_copy(idx_hbm.at[pl.ds(base, BPW), pl.ds(0, C0)], idx0_v)
        pltpu.sync_copy(idx_hbm.at[pl.ds(base, BPW), pl.ds(C0, C1)], idx1_v)

        sems = (sem0, sem1)

        def copies(b, p):
            return (
                pltpu.make_async_copy(
                    table_hbm.at[idx0_v.at[b]], rows0_v.at[p], sems[p]),
                pltpu.make_async_copy(
                    table_hbm.at[idx1_v.at[b]], rows1_v.at[p], sems[p]),
            )

        def start(b, p):
            for cp in copies(b, p):
                cp.start()

        def reduce_and_store(b, p):
            for cp in copies(b, p):
                cp.wait()
            accs = tuple(
                jnp.full((_LANES,), -jnp.inf, jnp.float32) for _ in range(ND))

            def rbody(rows):
                def f(r, accs):
                    return tuple(
                        jnp.maximum(accs[k], rows[p, r, pl.ds(k * _LANES, _LANES)])
                        for k in range(ND))
                return f

            accs = lax.fori_loop(0, C0, rbody(rows0_v), accs, unroll=8)
            accs = lax.fori_loop(0, C1, rbody(rows1_v), accs, unroll=8)
            for k in range(ND):
                pool_v[b, pl.ds(k * _LANES, _LANES)] = accs[k]

        NB2 = BPW // 2
        start(0, 0)

        def body(i, _):
            b0 = 2 * i
            start(b0 + 1, 1)
            reduce_and_store(b0, 0)

            @pl.when(i + 1 < NB2)
            def _():
                start(b0 + 2, 0)

            reduce_and_store(b0 + 1, 1)
            return 0

        lax.fori_loop(0, NB2, body, 0)

        pltpu.sync_copy(pool_v, out_hbm.at[pl.ds(base, BPW)])

    return pool


def _dense_softmax(pooled, w_pad, b_pad, n_out):
    B, D = pooled.shape
    BLK = 512

    def body(x_ref, w_ref, b_ref, o_ref):
        x = x_ref[...]
        logits = jnp.dot(x, w_ref[...], preferred_element_type=jnp.float32)
        logits = logits + b_ref[...]
        mask = lax.broadcasted_iota(jnp.int32, logits.shape, 1) < n_out
        masked = jnp.where(mask, logits, -jnp.inf)
        m = jnp.max(masked, axis=1, keepdims=True)
        e = jnp.where(mask, jnp.exp(masked - m), 0.0)
        o_ref[...] = e / jnp.sum(e, axis=1, keepdims=True)

    out = pl.pallas_call(
        body,
        grid=(B // BLK,),
        in_specs=[
            pl.BlockSpec((BLK, D), lambda i: (i, 0)),
            pl.BlockSpec((D, 128), lambda i: (0, 0)),
            pl.BlockSpec((1, 128), lambda i: (0, 0)),
        ],
        out_specs=pl.BlockSpec((BLK, 128), lambda i: (i, 0)),
        out_shape=jax.ShapeDtypeStruct((B, 128), jnp.float32),
    )(pooled, w_pad, b_pad)
    return out[:, :n_out]


def kernel(inputs, table, W_out, b_out):
    B, S = inputs.shape
    V, D = table.shape
    n_out = W_out.shape[1]

    idx = inputs.astype(jnp.int32)
    pooled = _make_pool_kernel(B, S, D)(idx, table)

    w_pad = jnp.zeros((D, 128), jnp.float32).at[:, :n_out].set(W_out)
    b_pad = jnp.zeros((1, 128), jnp.float32).at[0, :n_out].set(b_out)
    return _dense_softmax(pooled, w_pad, b_pad, n_out)


# speedup vs baseline: 14.2965x; 1.0182x over previous
import functools

import jax
import jax.numpy as jnp
from jax import lax
from jax.experimental import pallas as pl
from jax.experimental.pallas import tpu as pltpu
from jax.experimental.pallas import tpu_sc as plsc

_NC = 2
_NS = 16
_NW = _NC * _NS
_LANES = 16


def _make_pool_kernel(B, S, D):
    BPW = B // _NW
    C0 = 104
    C1 = S - C0
    BLANES = 2 * _LANES
    ND = D // BLANES

    mesh = plsc.VectorSubcoreMesh(core_axis_name="c", subcore_axis_name="s")

    @functools.partial(
        pl.kernel,
        mesh=mesh,
        out_type=jax.ShapeDtypeStruct((B, D), jnp.bfloat16),
        compiler_params=pltpu.CompilerParams(use_tc_tiling_on_sc=False),
        scratch_types=[
            pltpu.VMEM((BPW, C0), jnp.int32),
            pltpu.VMEM((BPW, C1), jnp.int32),
            pltpu.VMEM((2, C0, D), jnp.bfloat16),
            pltpu.VMEM((2, C1, D), jnp.bfloat16),
            pltpu.VMEM((BPW, D), jnp.bfloat16),
            pltpu.SemaphoreType.DMA,
            pltpu.SemaphoreType.DMA,
        ],
    )
    def pool(idx_hbm, table_hbm, out_hbm, idx0_v, idx1_v, rows0_v, rows1_v,
             pool_v, sem0, sem1):
        cid = lax.axis_index("c")
        sid = lax.axis_index("s")
        wid = sid * _NC + cid
        base = wid * BPW

        pltpu.sync_copy(idx_hbm.at[pl.ds(base, BPW), pl.ds(0, C0)], idx0_v)
        pltpu.sync_copy(idx_hbm.at[pl.ds(base, BPW), pl.ds(C0, C1)], idx1_v)

        sems = (sem0, sem1)

        def copies(b, p):
            return (
                pltpu.make_async_copy(
                    table_hbm.at[idx0_v.at[b]], rows0_v.at[p], sems[p]),
                pltpu.make_async_copy(
                    table_hbm.at[idx1_v.at[b]], rows1_v.at[p], sems[p]),
            )

        def start(b, p):
            for cp in copies(b, p):
                cp.start()

        def reduce_and_store(b, p):
            for cp in copies(b, p):
                cp.wait()
            accs = tuple(
                jnp.full((BLANES,), -jnp.inf, jnp.bfloat16) for _ in range(ND))

            def rbody(rows):
                def f(r, accs):
                    return tuple(
                        jnp.maximum(accs[k], rows[p, r, pl.ds(k * BLANES, BLANES)])
                        for k in range(ND))
                return f

            accs = lax.fori_loop(0, C0, rbody(rows0_v), accs, unroll=8)
            accs = lax.fori_loop(0, C1, rbody(rows1_v), accs, unroll=8)
            for k in range(ND):
                pool_v[b, pl.ds(k * BLANES, BLANES)] = accs[k]

        NB2 = BPW // 2
        start(0, 0)

        def body(i, _):
            b0 = 2 * i
            start(b0 + 1, 1)
            reduce_and_store(b0, 0)

            @pl.when(i + 1 < NB2)
            def _():
                start(b0 + 2, 0)

            reduce_and_store(b0 + 1, 1)
            return 0

        lax.fori_loop(0, NB2, body, 0)

        pltpu.sync_copy(pool_v, out_hbm.at[pl.ds(base, BPW)])

    return pool


def _dense_softmax(pooled, w_pad, b_pad, n_out):
    B, D = pooled.shape
    BLK = 512

    def body(x_ref, w_ref, b_ref, o_ref):
        x = x_ref[...].astype(jnp.float32)
        logits = jnp.dot(x, w_ref[...], preferred_element_type=jnp.float32)
        logits = logits + b_ref[...]
        mask = lax.broadcasted_iota(jnp.int32, logits.shape, 1) < n_out
        masked = jnp.where(mask, logits, -jnp.inf)
        m = jnp.max(masked, axis=1, keepdims=True)
        e = jnp.where(mask, jnp.exp(masked - m), 0.0)
        o_ref[...] = e / jnp.sum(e, axis=1, keepdims=True)

    out = pl.pallas_call(
        body,
        grid=(B // BLK,),
        in_specs=[
            pl.BlockSpec((BLK, D), lambda i: (i, 0)),
            pl.BlockSpec((D, 128), lambda i: (0, 0)),
            pl.BlockSpec((1, 128), lambda i: (0, 0)),
        ],
        out_specs=pl.BlockSpec((BLK, 128), lambda i: (i, 0)),
        out_shape=jax.ShapeDtypeStruct((B, 128), jnp.float32),
    )(pooled, w_pad, b_pad)
    return out[:, :n_out]


def kernel(inputs, table, W_out, b_out):
    B, S = inputs.shape
    V, D = table.shape
    n_out = W_out.shape[1]

    idx = inputs.astype(jnp.int32)
    pooled = _make_pool_kernel(B, S, D)(idx, table.astype(jnp.bfloat16))

    w_pad = jnp.zeros((D, 128), jnp.float32).at[:, :n_out].set(W_out)
    b_pad = jnp.zeros((1, 128), jnp.float32).at[0, :n_out].set(b_out)
    return _dense_softmax(pooled, w_pad, b_pad, n_out)
